# baseline (speedup 1.0000x reference)
import dataclasses

import jax
import jax.numpy as jnp
import numpy as np
from jax import lax
from jax.experimental import pallas as pl
from jax.experimental.pallas import tpu as pltpu
from jax.experimental.pallas import tpu_sc as plsc

BIT = 7
SIZE = 2 ** BIT
NB = 8
NP = 16384
NROWS = 16
L = 16
SENTINEL = (1 << 22) - 1
INV_CELL = np.float32(1.0 / 50.0)


def _encode_body(x_ref, flat_ref):
    x = x_ref[...]
    mn = jnp.min(x, axis=1, keepdims=True)
    q = ((x - mn) / INV_CELL).astype(jnp.int32)
    g = jnp.where(q >= SIZE, SIZE - 1, q)
    g0, g1, g2 = g[0:NB], g[NB:2 * NB], g[2 * NB:3 * NB]
    base = g2 * (SIZE * SIZE)
    flat_ref[...] = jnp.concatenate(
        [base + g1 * SIZE + g0,
         base + g0 * SIZE + g1],
        axis=0)


def _encode(pts):
    return pl.pallas_call(
        _encode_body,
        out_shape=jax.ShapeDtypeStruct((NROWS, NP), jnp.int32),
    )(pts)


def _radix_pass(cnt_v, kin, vin, kout, vout, shift, nbits, n, unroll):
    nbins = 1 << nbits
    dmask = nbins - 1

    @pl.loop(0, nbins, step=L, unroll=8)
    def _(j):
        cnt_v[pl.ds(j, L)] = jnp.zeros((L,), jnp.int32)

    @pl.loop(0, n, step=L, unroll=8 if unroll else None)
    def _(i):
        k = kin[pl.ds(i, L)]
        d = (k >> shift) & dmask
        counts, last = plsc.scan_count(d)
        plsc.addupdate_scatter(cnt_v, [d], counts, mask=last)

    def _scan(j, carry):
        v = cnt_v[pl.ds(j * L, L)]
        cs = plsc.cumsum(v)
        cnt_v[pl.ds(j * L, L)] = cs - v + carry
        return carry + jnp.sum(v)

    pl.loop(0, nbins // L, init_carry=jnp.int32(0))(_scan)

    @pl.loop(0, n, step=L, unroll=4 if unroll else None)
    def _(i):
        k = kin[pl.ds(i, L)]
        v = lax.iota(jnp.int32, L) + i if vin is None else vin[pl.ds(i, L)]
        d = (k >> shift) & dmask
        counts, last = plsc.scan_count(d)
        pos = plsc.load_gather(cnt_v, [d]) + counts - 1
        plsc.store_scatter(kout, [pos], k)
        plsc.store_scatter(vout, [pos], v)
        plsc.addupdate_scatter(cnt_v, [d], counts, mask=last)


def _sc_sort_body(flat_hbm, tmpl_hbm, order_hbm, rev_hbm,
                  b1, b2, b3, b4, b5, b6, cnt_v, sem):
    c = lax.axis_index("c")
    s = lax.axis_index("s")
    row = s * 2 + c

    @pl.when(s < NROWS // 2)
    def _():
        pltpu.sync_copy(flat_hbm.at[row], b1)

        _radix_pass(cnt_v, b1, None, b2, b3, 0, 11, NP, True)
        _radix_pass(cnt_v, b2, b3, b4, b5, 11, 10, NP, True)

        @pl.loop(0, NP, step=L, unroll=8)
        def _(i):
            b1[pl.ds(i, L)] = lax.iota(jnp.int32, L) + i

        def _dedup(i, carry):
            prev, tot = carry
            f = b4[pl.ds(i, L)]
            counts, _ = plsc.scan_count(f)
            is_new = (counts == 1) & (f != prev)
            inc = is_new.astype(jnp.int32)
            u = tot + plsc.cumsum(inc) - 1
            idx = b5[pl.ds(i, L)]
            b5[pl.ds(i, L)] = (u << 14) | idx
            plsc.store_scatter(b1, [u], f, mask=is_new)
            plsc.store_scatter(b6, [u], lax.iota(jnp.int32, L) + i, mask=is_new)
            return (jnp.max(f), tot + jnp.sum(inc))

        _, num_u = pl.loop(0, NP, step=L,
                           init_carry=(jnp.int32(-1), jnp.int32(0)))(_dedup)

        lane0 = lax.iota(jnp.int32, L) == 0
        plsc.store_scatter(b6, [jnp.zeros((L,), jnp.int32) + num_u],
                           jnp.full((L,), NP, jnp.int32), mask=lane0)

        nwaves = (num_u + 1023) // 1024

        @pl.loop(0, nwaves)
        def _(w):
            base = w * 1024
            cps = [
                pltpu.async_copy(
                    tmpl_hbm.at[b1.at[pl.ds(base + t * 128, 128)]],
                    b2.at[pl.ds(base + t * 128, 128)], sem)
                for t in range(8)
            ]
            for cp in cps:
                cp.wait()

        upad = ((num_u + L - 1) // L) * L

        @pl.loop(0, upad, step=L)
        def _(k):
            cvec = b2[pl.ds(k, L)]
            real = (lax.iota(jnp.int32, L) + k) < num_u
            b2[pl.ds(k, L)] = jnp.where(real, cvec, jnp.int32(SENTINEL))

        _radix_pass(cnt_v, b2, None, b3, b4, 0, 11, upad, False)
        _radix_pass(cnt_v, b3, b4, b2, b1, 11, 11, upad, False)

        def _lens(k, carry):
            u = b1[pl.ds(k, L)]
            real = (lax.iota(jnp.int32, L) + k) < num_u
            rs = plsc.load_gather(b6, [u])
            rsn = plsc.load_gather(b6, [u + 1])
            ln = jnp.where(real, rsn - rs, 0)
            cs = plsc.cumsum(ln)
            plsc.store_scatter(b4, [u], cs - ln + carry, mask=real)
            return carry + jnp.sum(ln)

        pl.loop(0, upad, step=L, init_carry=jnp.int32(0))(_lens)

        @pl.loop(0, NP, step=L, unroll=4)
        def _(i):
            w = b5[pl.ds(i, L)]
            u = w >> 14
            idx = w & (NP - 1)
            start = plsc.load_gather(b4, [u])
            rs = plsc.load_gather(b6, [u])
            pos = start + (lax.iota(jnp.int32, L) + i) - rs
            plsc.store_scatter(b2, [pos], idx)
            plsc.store_scatter(b3, [idx], pos)

        pltpu.sync_copy(b2, order_hbm.at[row])
        pltpu.sync_copy(b3, rev_hbm.at[row])


def _sc_sort(flat, template):
    mesh = plsc.VectorSubcoreMesh(core_axis_name="c", subcore_axis_name="s")
    cp = pltpu.CompilerParams()
    if "needs_layout_passes" in pltpu.CompilerParams.__dataclass_fields__:
        cp = dataclasses.replace(cp, needs_layout_passes=False)
    f = pl.kernel(
        _sc_sort_body,
        out_type=(jax.ShapeDtypeStruct((NROWS, NP), jnp.int32),
                  jax.ShapeDtypeStruct((NROWS, NP), jnp.int32)),
        mesh=mesh,
        scratch_types=[
            pltpu.VMEM((NP,), jnp.int32),
            pltpu.VMEM((NP,), jnp.int32),
            pltpu.VMEM((NP,), jnp.int32),
            pltpu.VMEM((NP,), jnp.int32),
            pltpu.VMEM((NP,), jnp.int32),
            pltpu.VMEM((NP + L,), jnp.int32),
            pltpu.VMEM((2048,), jnp.int32),
            pltpu.SemaphoreType.DMA,
        ],
        compiler_params=cp,
    )
    return f(flat, template)


def kernel(points, template):
    pts = jnp.transpose(points, (2, 0, 1)).reshape(3 * NB, NP)
    flat = _encode(pts)
    order, rev = _sc_sort(flat, template)
    return (order.reshape(2, NB, NP), rev.reshape(2, NB, NP))


# speedup vs baseline: 1.0057x; 1.0057x over previous
import dataclasses

import jax
import jax.numpy as jnp
import numpy as np
from jax import lax
from jax.experimental import pallas as pl
from jax.experimental.pallas import tpu as pltpu
from jax.experimental.pallas import tpu_sc as plsc

BIT = 7
SIZE = 2 ** BIT
NB = 8
NP = 16384
NROWS = 16
L = 16
SENTINEL = (1 << 22) - 1
INV_CELL = np.float32(1.0 / 50.0)


def _encode_body(x_ref, flat_ref):
    x = x_ref[...]
    mn = jnp.min(x, axis=1, keepdims=True)
    q = ((x - mn) / INV_CELL).astype(jnp.int32)
    g = jnp.where(q >= SIZE, SIZE - 1, q)
    g0, g1, g2 = g[0:NB], g[NB:2 * NB], g[2 * NB:3 * NB]
    base = g2 * (SIZE * SIZE)
    flat_ref[...] = jnp.concatenate(
        [base + g1 * SIZE + g0,
         base + g0 * SIZE + g1],
        axis=0)


def _encode(pts):
    return pl.pallas_call(
        _encode_body,
        out_shape=jax.ShapeDtypeStruct((NROWS, NP), jnp.int32),
    )(pts)


def _radix_pass(cnt_v, kin, vin, kout, vout, shift, nbits, n, unroll):
    nbins = 1 << nbits
    dmask = nbins - 1

    @pl.loop(0, nbins, step=L, unroll=8)
    def _(j):
        cnt_v[pl.ds(j, L)] = jnp.zeros((L,), jnp.int32)

    @pl.loop(0, n, step=L, unroll=8 if unroll else None)
    def _(i):
        k = kin[pl.ds(i, L)]
        d = (k >> shift) & dmask
        counts, last = plsc.scan_count(d)
        plsc.addupdate_scatter(cnt_v, [d], counts, mask=last)

    def _scan(j, carry):
        v = cnt_v[pl.ds(j * L, L)]
        cs = plsc.cumsum(v)
        cnt_v[pl.ds(j * L, L)] = cs - v + carry
        return carry + jnp.sum(v)

    pl.loop(0, nbins // L, init_carry=jnp.int32(0))(_scan)

    @pl.loop(0, n, step=L, unroll=4 if unroll else None)
    def _(i):
        k = kin[pl.ds(i, L)]
        v = lax.iota(jnp.int32, L) + i if vin is None else vin[pl.ds(i, L)]
        d = (k >> shift) & dmask
        counts, last = plsc.scan_count(d)
        pos = plsc.load_gather(cnt_v, [d]) + counts - 1
        plsc.store_scatter(kout, [pos], k)
        plsc.store_scatter(vout, [pos], v)
        plsc.addupdate_scatter(cnt_v, [d], counts, mask=last)


def _sc_sort_body(flat_hbm, tmpl_hbm, order_hbm, rev_hbm,
                  b1, b2, b3, b4, b5, b6, cnt_v, sem):
    c = lax.axis_index("c")
    s = lax.axis_index("s")
    row = s * 2 + c

    @pl.when(s < NROWS // 2)
    def _():
        pltpu.sync_copy(flat_hbm.at[row], b1)

        _radix_pass(cnt_v, b1, None, b2, b3, 0, 11, NP, True)
        _radix_pass(cnt_v, b2, b3, b4, b5, 11, 10, NP, True)

        def _dedup(i, carry):
            prev, tot = carry
            f = b4[pl.ds(i, L)]
            counts, _ = plsc.scan_count(f)
            is_new = (counts == 1) & (f != prev)
            inc = is_new.astype(jnp.int32)
            u = tot + plsc.cumsum(inc) - 1
            idx = b5[pl.ds(i, L)]
            b5[pl.ds(i, L)] = (u << 14) | idx
            plsc.store_scatter(b1, [u], f, mask=is_new)
            plsc.store_scatter(b6, [u], lax.iota(jnp.int32, L) + i, mask=is_new)
            return (jnp.max(f), tot + jnp.sum(inc))

        _, num_u = pl.loop(0, NP, step=L,
                           init_carry=(jnp.int32(-1), jnp.int32(0)))(_dedup)

        lane0 = lax.iota(jnp.int32, L) == 0
        plsc.store_scatter(b6, [jnp.zeros((L,), jnp.int32) + num_u],
                           jnp.full((L,), NP, jnp.int32), mask=lane0)

        nwaves = (num_u + 1023) // 1024
        tail0 = (num_u // L) * L

        @pl.loop(tail0, nwaves * 1024, step=L)
        def _(k):
            iot = lax.iota(jnp.int32, L) + k
            plsc.store_scatter(b1, [iot], iot, mask=iot >= num_u)


        @pl.loop(0, nwaves)
        def _(w):
            base = w * 1024
            cps = [
                pltpu.async_copy(
                    tmpl_hbm.at[b1.at[pl.ds(base + t * 128, 128)]],
                    b2.at[pl.ds(base + t * 128, 128)], sem)
                for t in range(8)
            ]
            for cp in cps:
                cp.wait()

        upad = ((num_u + L - 1) // L) * L

        @pl.loop(0, upad, step=L)
        def _(k):
            cvec = b2[pl.ds(k, L)]
            real = (lax.iota(jnp.int32, L) + k) < num_u
            b2[pl.ds(k, L)] = jnp.where(real, cvec, jnp.int32(SENTINEL))

        _radix_pass(cnt_v, b2, None, b3, b4, 0, 11, upad, False)
        _radix_pass(cnt_v, b3, b4, b2, b1, 11, 11, upad, False)

        def _lens(k, carry):
            u = b1[pl.ds(k, L)]
            real = (lax.iota(jnp.int32, L) + k) < num_u
            rs = plsc.load_gather(b6, [u])
            rsn = plsc.load_gather(b6, [u + 1])
            ln = jnp.where(real, rsn - rs, 0)
            cs = plsc.cumsum(ln)
            plsc.store_scatter(b4, [u], cs - ln + carry - rs, mask=real)
            return carry + jnp.sum(ln)

        pl.loop(0, upad, step=L, init_carry=jnp.int32(0))(_lens)

        @pl.loop(0, NP, step=L, unroll=8)
        def _(i):
            w = b5[pl.ds(i, L)]
            u = w >> 14
            idx = w & (NP - 1)
            pos = plsc.load_gather(b4, [u]) + (lax.iota(jnp.int32, L) + i)
            plsc.store_scatter(b2, [pos], idx)
            plsc.store_scatter(b3, [idx], pos)

        pltpu.sync_copy(b2, order_hbm.at[row])
        pltpu.sync_copy(b3, rev_hbm.at[row])


def _sc_sort(flat, template):
    mesh = plsc.VectorSubcoreMesh(core_axis_name="c", subcore_axis_name="s")
    cp = pltpu.CompilerParams()
    if "needs_layout_passes" in pltpu.CompilerParams.__dataclass_fields__:
        cp = dataclasses.replace(cp, needs_layout_passes=False)
    f = pl.kernel(
        _sc_sort_body,
        out_type=(jax.ShapeDtypeStruct((NROWS, NP), jnp.int32),
                  jax.ShapeDtypeStruct((NROWS, NP), jnp.int32)),
        mesh=mesh,
        scratch_types=[
            pltpu.VMEM((NP,), jnp.int32),
            pltpu.VMEM((NP,), jnp.int32),
            pltpu.VMEM((NP,), jnp.int32),
            pltpu.VMEM((NP,), jnp.int32),
            pltpu.VMEM((NP,), jnp.int32),
            pltpu.VMEM((NP + L,), jnp.int32),
            pltpu.VMEM((2048,), jnp.int32),
            pltpu.SemaphoreType.DMA,
        ],
        compiler_params=cp,
    )
    return f(flat, template)


def kernel(points, template):
    pts = jnp.transpose(points, (2, 0, 1)).reshape(3 * NB, NP)
    flat = _encode(pts)
    order, rev = _sc_sort(flat, template)
    return (order.reshape(2, NB, NP), rev.reshape(2, NB, NP))


# speedup vs baseline: 1.1591x; 1.1526x over previous
import dataclasses

import jax
import jax.numpy as jnp
import numpy as np
from jax import lax
from jax.experimental import pallas as pl
from jax.experimental.pallas import tpu as pltpu
from jax.experimental.pallas import tpu_sc as plsc

BIT = 7
SIZE = 2 ** BIT
NB = 8
NP = 16384
NH = NP // 2
NROWS = 16
L = 16
SENTINEL = (1 << 22) - 1
INV_CELL = np.float32(1.0 / 50.0)


def _encode_body(x_ref, flat_ref):
    x = x_ref[...]
    mn = jnp.min(x, axis=1, keepdims=True)
    q = ((x - mn) / INV_CELL).astype(jnp.int32)
    g = jnp.where(q >= SIZE, SIZE - 1, q)
    g0, g1, g2 = g[0:NB], g[NB:2 * NB], g[2 * NB:3 * NB]
    base = g2 * (SIZE * SIZE)
    flat_ref[...] = jnp.concatenate(
        [base + g1 * SIZE + g0,
         base + g0 * SIZE + g1],
        axis=0)


def _encode(pts):
    return pl.pallas_call(
        _encode_body,
        out_shape=jax.ShapeDtypeStruct((NROWS, NP), jnp.int32),
    )(pts)


def _radix_pass(cnt_v, kin, vin, kout, vout, shift, nbits, n, unroll,
                idx_base=0):
    nbins = 1 << nbits
    dmask = nbins - 1

    @pl.loop(0, nbins, step=L, unroll=8)
    def _(j):
        cnt_v[pl.ds(j, L)] = jnp.zeros((L,), jnp.int32)

    @pl.loop(0, n, step=L, unroll=8 if unroll else None)
    def _(i):
        k = kin[pl.ds(i, L)]
        d = (k >> shift) & dmask
        counts, last = plsc.scan_count(d)
        plsc.addupdate_scatter(cnt_v, [d], counts, mask=last)

    def _scan(j, carry):
        v = cnt_v[pl.ds(j * L, L)]
        cs = plsc.cumsum(v)
        cnt_v[pl.ds(j * L, L)] = cs - v + carry
        return carry + jnp.sum(v)

    pl.loop(0, nbins // L, init_carry=jnp.int32(0))(_scan)

    @pl.loop(0, n, step=L, unroll=4 if unroll else None)
    def _(i):
        k = kin[pl.ds(i, L)]
        v = (lax.iota(jnp.int32, L) + (i + idx_base) if vin is None
             else vin[pl.ds(i, L)])
        d = (k >> shift) & dmask
        counts, last = plsc.scan_count(d)
        pos = plsc.load_gather(cnt_v, [d]) + counts - 1
        plsc.store_scatter(kout, [pos], k)
        plsc.store_scatter(vout, [pos], v)
        plsc.addupdate_scatter(cnt_v, [d], counts, mask=last)


def _sc_sort_body(flat_hbm, tmpl_hbm, order_hbm, rev_hbm,
                  b1, b2, b3, b4, b5, b6, cnt_v, oex, pc, pe, rev_loc,
                  sh_codes, sh_excl, sh_pack, sem):
    c = lax.axis_index("c")
    s = lax.axis_index("s")
    half = s & 1
    hbase = half * NH
    wrow = (8 * c + (s >> 1)) * 2 + half
    iot = lax.iota(jnp.int32, L)

    pltpu.sync_copy(flat_hbm.at[wrow], b1)

    _radix_pass(cnt_v, b1, None, b2, b3, 0, 11, NH, True, idx_base=hbase)
    _radix_pass(cnt_v, b2, b3, b4, b5, 11, 10, NH, True)

    def _dedup(i, carry):
        prev, tot = carry
        f = b4[pl.ds(i, L)]
        counts, _ = plsc.scan_count(f)
        is_new = (counts == 1) & (f != prev)
        inc = is_new.astype(jnp.int32)
        u = tot + plsc.cumsum(inc) - 1
        gidx = b5[pl.ds(i, L)]
        b5[pl.ds(i, L)] = (u << 14) | gidx
        plsc.store_scatter(b1, [u], f, mask=is_new)
        plsc.store_scatter(b6, [u], iot + (i + hbase), mask=is_new)
        return (jnp.max(f), tot + jnp.sum(inc))

    _, num_u = pl.loop(0, NH, step=L,
                       init_carry=(jnp.int32(-1), jnp.int32(0)))(_dedup)

    lane0 = iot == 0
    plsc.store_scatter(b6, [jnp.zeros((L,), jnp.int32) + num_u],
                       jnp.full((L,), NH, jnp.int32) + hbase, mask=lane0)

    nwaves = (num_u + 1023) // 1024
    tail0 = (num_u // L) * L

    @pl.loop(tail0, nwaves * 1024, step=L)
    def _(k):
        kv = iot + k
        plsc.store_scatter(b1, [kv], kv, mask=kv >= num_u)

    @pl.loop(0, nwaves)
    def _(w):
        base = w * 1024
        cps = [
            pltpu.async_copy(
                tmpl_hbm.at[b1.at[pl.ds(base + t * 128, 128)]],
                b2.at[pl.ds(base + t * 128, 128)], sem)
            for t in range(8)
        ]
        for cp in cps:
            cp.wait()

    @pl.loop(0, NH, step=L, unroll=4)
    def _(k):
        cvec = b2[pl.ds(k, L)]
        b2[pl.ds(k, L)] = jnp.where(iot + k < num_u, cvec,
                                    jnp.int32(SENTINEL))

    upad = ((num_u + L - 1) // L) * L

    _radix_pass(cnt_v, b2, None, b3, b4, 0, 11, upad, False)
    _radix_pass(cnt_v, b3, b4, b2, b1, 11, 11, upad, False)

    def _lens(k, carry):
        u = b1[pl.ds(k, L)]
        real = iot + k < num_u
        rs = plsc.load_gather(b6, [u])
        rsn = plsc.load_gather(b6, [u + 1])
        ln = jnp.where(real, rsn - rs, 0)
        cs = plsc.cumsum(ln)
        oex[pl.ds(k, L)] = cs - ln + carry
        return carry + jnp.sum(ln)

    pl.loop(0, upad, step=L, init_carry=jnp.int32(0))(_lens)
    plsc.store_scatter(oex, [jnp.zeros((L,), jnp.int32) + num_u],
                       jnp.full((L,), NH, jnp.int32), mask=lane0)

    pltpu.sync_copy(b2, sh_codes.at[s])
    pltpu.sync_copy(oex, sh_excl.at[s])
    plsc.subcore_barrier()
    part = s ^ 1
    pltpu.sync_copy(sh_codes.at[part], pc.at[pl.ds(0, NH)])
    pltpu.sync_copy(sh_excl.at[part], pe)

    @pl.loop(0, upad, step=L)
    def _(k):
        cc = b2[pl.ds(k, L)]
        real = iot + k < num_u
        lo = jnp.zeros((L,), jnp.int32)
        hi = jnp.full((L,), NH, jnp.int32)
        for _step in range(14):
            upd = lo < hi
            mid = (lo + hi) >> 1
            less = plsc.load_gather(pc, [mid]) < cc
            lo = jnp.where(upd & less, mid + 1, lo)
            hi = jnp.where(upd & ~less, mid, hi)
        eq = (plsc.load_gather(pc, [lo]) == cc) & (half == 1)
        other = jnp.where(eq, plsc.load_gather(pe, [lo + 1]),
                          plsc.load_gather(pe, [lo]))
        gstart = oex[pl.ds(k, L)] + other
        u = b1[pl.ds(k, L)]
        delta = gstart - plsc.load_gather(b6, [u])
        plsc.store_scatter(b4, [u], delta, mask=real)

    @pl.loop(0, NH, step=L, unroll=4)
    def _(i):
        w = b5[pl.ds(i, L)]
        u = w >> 14
        gidx = w & (NP - 1)
        pos = plsc.load_gather(b4, [u]) + (iot + (i + hbase))
        b2[pl.ds(i, L)] = (pos << 14) | gidx
        plsc.store_scatter(rev_loc, [gidx - hbase], pos)

    pltpu.sync_copy(b2, sh_pack.at[s])
    plsc.subcore_barrier()
    pltpu.sync_copy(sh_pack.at[part], b3)

    for src in (b2, b3):
        @pl.loop(0, NH, step=L, unroll=4)
        def _(i):
            w = src[pl.ds(i, L)]
            pos = w >> 14
            gidx = w & (NP - 1)
            plsc.store_scatter(b4, [pos & (NH - 1)], gidx,
                               mask=(pos >> 13) == half)

    pltpu.sync_copy(b4, order_hbm.at[wrow])
    pltpu.sync_copy(rev_loc, rev_hbm.at[wrow])


def _sc_sort(flat, template):
    mesh = plsc.VectorSubcoreMesh(core_axis_name="c", subcore_axis_name="s")
    cp = pltpu.CompilerParams()
    if "needs_layout_passes" in pltpu.CompilerParams.__dataclass_fields__:
        cp = dataclasses.replace(cp, needs_layout_passes=False)
    f = pl.kernel(
        _sc_sort_body,
        out_type=(jax.ShapeDtypeStruct((2 * NROWS, NH), jnp.int32),
                  jax.ShapeDtypeStruct((2 * NROWS, NH), jnp.int32)),
        mesh=mesh,
        scratch_types=[
            pltpu.VMEM((NH,), jnp.int32),
            pltpu.VMEM((NH,), jnp.int32),
            pltpu.VMEM((NH,), jnp.int32),
            pltpu.VMEM((NH,), jnp.int32),
            pltpu.VMEM((NH,), jnp.int32),
            pltpu.VMEM((NH + L,), jnp.int32),
            pltpu.VMEM((2048,), jnp.int32),
            pltpu.VMEM((NH + L,), jnp.int32),
            pltpu.VMEM((NH + L,), jnp.int32),
            pltpu.VMEM((NH + L,), jnp.int32),
            pltpu.VMEM((NH,), jnp.int32),
            pltpu.VMEM_SHARED((16, NH), jnp.int32),
            pltpu.VMEM_SHARED((16, NH + L), jnp.int32),
            pltpu.VMEM_SHARED((16, NH), jnp.int32),
            pltpu.SemaphoreType.DMA,
        ],
        compiler_params=cp,
    )
    return f(flat, template)


def kernel(points, template):
    pts = jnp.transpose(points, (2, 0, 1)).reshape(3 * NB, NP)
    flat = _encode(pts)
    order, rev = _sc_sort(flat.reshape(2 * NROWS, NH), template)
    return (order.reshape(2, NB, NP), rev.reshape(2, NB, NP))


# speedup vs baseline: 1.4378x; 1.2404x over previous
import dataclasses

import jax
import jax.numpy as jnp
import numpy as np
from jax import lax
from jax.experimental import pallas as pl
from jax.experimental.pallas import tpu as pltpu
from jax.experimental.pallas import tpu_sc as plsc

BIT = 7
SIZE = 2 ** BIT
NB = 8
NP = 16384
NH = NP // 2
NROWS = 16
L = 16
SENTINEL = (1 << 22) - 1
INV_CELL = np.float32(1.0 / 50.0)


def _encode_body(x_ref, flat_ref):
    x = x_ref[...]
    mn = jnp.min(x, axis=1, keepdims=True)
    q = ((x - mn) / INV_CELL).astype(jnp.int32)
    g = jnp.where(q >= SIZE, SIZE - 1, q)
    g0, g1, g2 = g[0:NB], g[NB:2 * NB], g[2 * NB:3 * NB]
    base = g2 * (SIZE * SIZE)
    flat_ref[...] = jnp.concatenate(
        [base + g1 * SIZE + g0,
         base + g0 * SIZE + g1],
        axis=0)


def _encode(pts):
    return pl.pallas_call(
        _encode_body,
        out_shape=jax.ShapeDtypeStruct((NROWS, NP), jnp.int32),
    )(pts)


def _radix_pass(cnt_v, kin, vin, kout, vout, shift, nbits, n, unroll,
                idx_base=0, hash_fold=False, shiftbuf=None):
    nbins = 1 << nbits
    dmask = nbins - 1

    def digit(k):
        return ((k ^ (k >> 10)) if hash_fold else (k >> shift)) & dmask

    @pl.loop(0, nbins, step=L, unroll=8)
    def _(j):
        cnt_v[pl.ds(j, L)] = jnp.zeros((L,), jnp.int32)

    @pl.loop(0, n, step=L, unroll=8 if unroll else None)
    def _(i):
        d = digit(kin[pl.ds(i, L)])
        counts, last = plsc.scan_count(d)
        plsc.addupdate_scatter(cnt_v, [d], counts, mask=last)

    def _scan(j, carry):
        v = cnt_v[pl.ds(j * L, L)]
        cs = plsc.cumsum(v)
        cnt_v[pl.ds(j * L, L)] = cs - v + carry
        return carry + jnp.sum(v)

    pl.loop(0, nbins // L, init_carry=jnp.int32(0))(_scan)

    @pl.loop(0, n, step=L, unroll=4 if unroll else None)
    def _(i):
        k = kin[pl.ds(i, L)]
        v = (lax.iota(jnp.int32, L) + (i + idx_base) if vin is None
             else vin[pl.ds(i, L)])
        d = digit(k)
        counts, last = plsc.scan_count(d)
        pos = plsc.load_gather(cnt_v, [d]) + counts - 1
        plsc.store_scatter(kout, [pos], k)
        plsc.store_scatter(vout, [pos], v)
        if shiftbuf is not None:
            plsc.store_scatter(shiftbuf, [pos + 1], k)
        plsc.addupdate_scatter(cnt_v, [d], counts, mask=last)


def _sc_sort_body(flat_hbm, tmpl_hbm, order_hbm, rev_hbm,
                  b1, b2, b3, b4, b5, b6, cnt_v, oex, pc, pe, rev_loc,
                  sh_codes, sh_excl, sh_pack, sem):
    c = lax.axis_index("c")
    s = lax.axis_index("s")
    half = s & 1
    hbase = half * NH
    wrow = (8 * c + (s >> 1)) * 2 + half
    iot = lax.iota(jnp.int32, L)

    pltpu.sync_copy(flat_hbm.at[wrow], b1)

    plsc.store_scatter(b4, [jnp.zeros((L,), jnp.int32)],
                       jnp.full((L,), -1, jnp.int32), mask=iot == 0)
    _radix_pass(cnt_v, b1, None, b2, b3, 0, 11, NH, True, idx_base=hbase,
                hash_fold=True, shiftbuf=b4)

    def _dedup(i, tot):
        f = b2[pl.ds(i, L)]
        is_new = f != b4[pl.ds(i, L)]
        inc = is_new.astype(jnp.int32)
        u = tot + plsc.cumsum(inc) - 1
        b3[pl.ds(i, L)] = (u << 14) | b3[pl.ds(i, L)]
        plsc.store_scatter(b1, [u], f, mask=is_new)
        plsc.store_scatter(b6, [u], iot + (i + hbase), mask=is_new)
        return tot + jnp.sum(inc)

    num_u = pl.loop(0, NH, step=L, init_carry=jnp.int32(0))(_dedup)

    lane0 = iot == 0
    plsc.store_scatter(b6, [jnp.zeros((L,), jnp.int32) + num_u],
                       jnp.full((L,), NH, jnp.int32) + hbase, mask=lane0)

    nwaves = (num_u + 1023) // 1024
    tail0 = (num_u // L) * L

    @pl.loop(tail0, nwaves * 1024, step=L)
    def _(k):
        kv = iot + k
        plsc.store_scatter(b1, [kv], kv, mask=kv >= num_u)

    @pl.loop(0, nwaves)
    def _(w):
        base = w * 1024
        cps = [
            pltpu.async_copy(
                tmpl_hbm.at[b1.at[pl.ds(base + t * 128, 128)]],
                b2.at[pl.ds(base + t * 128, 128)], sem)
            for t in range(8)
        ]
        for cp in cps:
            cp.wait()

    @pl.loop(0, NH, step=L, unroll=4)
    def _(k):
        cvec = b2[pl.ds(k, L)]
        b2[pl.ds(k, L)] = jnp.where(iot + k < num_u, cvec,
                                    jnp.int32(SENTINEL))

    upad = ((num_u + L - 1) // L) * L

    _radix_pass(cnt_v, b2, None, b5, b4, 0, 11, upad, False)
    _radix_pass(cnt_v, b5, b4, b2, b1, 11, 11, upad, False)

    def _lens(k, carry):
        u = b1[pl.ds(k, L)]
        real = iot + k < num_u
        rs = plsc.load_gather(b6, [u])
        rsn = plsc.load_gather(b6, [u + 1])
        ln = jnp.where(real, rsn - rs, 0)
        cs = plsc.cumsum(ln)
        oex[pl.ds(k, L)] = cs - ln + carry
        return carry + jnp.sum(ln)

    pl.loop(0, upad, step=L, init_carry=jnp.int32(0))(_lens)
    plsc.store_scatter(oex, [jnp.zeros((L,), jnp.int32) + num_u],
                       jnp.full((L,), NH, jnp.int32), mask=lane0)

    pltpu.sync_copy(b2, sh_codes.at[s])
    pltpu.sync_copy(oex, sh_excl.at[s])
    plsc.subcore_barrier()
    part = s ^ 1
    pltpu.sync_copy(sh_codes.at[part], pc.at[pl.ds(0, NH)])
    pltpu.sync_copy(sh_excl.at[part], pe)

    @pl.loop(0, upad, step=L)
    def _(k):
        tgt = b2[pl.ds(k, L)] + half
        real = iot + k < num_u
        lo = jnp.zeros((L,), jnp.int32)
        hi = jnp.full((L,), NH, jnp.int32)
        for _step in range(14):
            upd = lo < hi
            mid = (lo + hi) >> 1
            less = plsc.load_gather(pc, [mid]) < tgt
            lo = jnp.where(upd & less, mid + 1, lo)
            hi = jnp.where(upd & ~less, mid, hi)
        gstart = oex[pl.ds(k, L)] + plsc.load_gather(pe, [lo])
        u = b1[pl.ds(k, L)]
        delta = gstart - plsc.load_gather(b6, [u])
        plsc.store_scatter(b4, [u], delta, mask=real)

    @pl.loop(0, NH, step=L, unroll=4)
    def _(i):
        w = b3[pl.ds(i, L)]
        u = w >> 14
        gidx = w & (NP - 1)
        pos = plsc.load_gather(b4, [u]) + (iot + (i + hbase))
        b2[pl.ds(i, L)] = (pos << 14) | gidx
        plsc.store_scatter(b1, [pos & (NH - 1)], gidx,
                           mask=(pos >> 13) == half)
        plsc.store_scatter(rev_loc, [gidx - hbase], pos)

    pltpu.sync_copy(b2, sh_pack.at[s])
    plsc.subcore_barrier()
    pltpu.sync_copy(sh_pack.at[part], b5)

    @pl.loop(0, NH, step=L, unroll=4)
    def _(i):
        w = b5[pl.ds(i, L)]
        pos = w >> 14
        gidx = w & (NP - 1)
        plsc.store_scatter(b1, [pos & (NH - 1)], gidx,
                           mask=(pos >> 13) == half)

    pltpu.sync_copy(b1, order_hbm.at[wrow])
    pltpu.sync_copy(rev_loc, rev_hbm.at[wrow])


def _sc_sort(flat, template):
    mesh = plsc.VectorSubcoreMesh(core_axis_name="c", subcore_axis_name="s")
    cp = pltpu.CompilerParams()
    if "needs_layout_passes" in pltpu.CompilerParams.__dataclass_fields__:
        cp = dataclasses.replace(cp, needs_layout_passes=False)
    f = pl.kernel(
        _sc_sort_body,
        out_type=(jax.ShapeDtypeStruct((2 * NROWS, NH), jnp.int32),
                  jax.ShapeDtypeStruct((2 * NROWS, NH), jnp.int32)),
        mesh=mesh,
        scratch_types=[
            pltpu.VMEM((NH,), jnp.int32),
            pltpu.VMEM((NH,), jnp.int32),
            pltpu.VMEM((NH,), jnp.int32),
            pltpu.VMEM((NH + L,), jnp.int32),
            pltpu.VMEM((NH,), jnp.int32),
            pltpu.VMEM((NH + L,), jnp.int32),
            pltpu.VMEM((2048,), jnp.int32),
            pltpu.VMEM((NH + L,), jnp.int32),
            pltpu.VMEM((NH + L,), jnp.int32),
            pltpu.VMEM((NH + L,), jnp.int32),
            pltpu.VMEM((NH,), jnp.int32),
            pltpu.VMEM_SHARED((16, NH), jnp.int32),
            pltpu.VMEM_SHARED((16, NH + L), jnp.int32),
            pltpu.VMEM_SHARED((16, NH), jnp.int32),
            pltpu.SemaphoreType.DMA,
        ],
        compiler_params=cp,
    )
    return f(flat, template)


def kernel(points, template):
    pts = jnp.transpose(points, (2, 0, 1)).reshape(3 * NB, NP)
    flat = _encode(pts)
    order, rev = _sc_sort(flat.reshape(2 * NROWS, NH), template)
    return (order.reshape(2, NB, NP), rev.reshape(2, NB, NP))
